# baseline (speedup 1.0000x reference)
import functools

import jax
import jax.numpy as jnp
import numpy as np
from jax import lax
from jax.experimental import pallas as pl
from jax.experimental.pallas import tpu as pltpu
from jax.experimental.pallas import tpu_sc as plsc

VOCAB = 1000000
DIM = 64
MAX_LEN = 256
B = 4096
S = 200

NUM_CORES = 2
NUM_SUBCORES = 16
NW = NUM_CORES * NUM_SUBCORES
BW = B // NW
NJ = BW // 16


def _sinusoidal_pe(max_len, dim):
    pos = np.arange(max_len, dtype=np.float32)[:, None]
    i = np.arange(0, dim, 2, dtype=np.float32)[None, :]
    angle = pos / np.power(10000.0, i / dim)
    pe = np.zeros((max_len, dim), dtype=np.float32)
    pe[:, 0::2] = np.sin(angle)
    pe[:, 1::2] = np.cos(angle)
    return pe


_PE = _sinusoidal_pe(MAX_LEN, DIM)[:S, :]
_PE_SPLAT = np.repeat(_PE, 16, axis=1).reshape(S, DIM * 16)


@functools.partial(
    pl.kernel,
    mesh=plsc.VectorSubcoreMesh(core_axis_name="c", subcore_axis_name="s"),
    out_type=jax.ShapeDtypeStruct((S, DIM, B), jnp.float32),
    compiler_params=pltpu.CompilerParams(
        use_tc_tiling_on_sc=True, needs_layout_passes=False
    ),
    scratch_types=[
        pltpu.VMEM((S, BW), jnp.int32),
        pltpu.VMEM((4, BW), jnp.int32),
        pltpu.VMEM((4, BW), jnp.int32),
        pltpu.VMEM((BW, 128), jnp.float32),
        pltpu.VMEM((BW, 128), jnp.float32),
        pltpu.VMEM((BW, 128), jnp.float32),
        pltpu.VMEM((BW, 128), jnp.float32),
        pltpu.VMEM((DIM, BW), jnp.float32),
        pltpu.VMEM((DIM, BW), jnp.float32),
        pltpu.VMEM((4, DIM * 16), jnp.float32),
        pltpu.SemaphoreType.DMA,
        pltpu.SemaphoreType.DMA,
        pltpu.SemaphoreType.DMA,
        pltpu.SemaphoreType.DMA,
        pltpu.SemaphoreType.DMA,
        pltpu.SemaphoreType.DMA,
    ],
)
def _emb(idx_hbm, t2_hbm, pe_hbm, out_hbm,
         idx_v, pidx_v, cb_v, g0, g1, g2, g3, o0, o1, psplat_v,
         gsem0, gsem1, gsem2, gsem3, osem0, osem1):
    wid = lax.axis_index("s") * NUM_CORES + lax.axis_index("c")
    g = (g0, g1, g2, g3)
    o = (o0, o1)
    gsem = (gsem0, gsem1, gsem2, gsem3)
    osem = (osem0, osem1)

    pltpu.sync_copy(idx_hbm.at[:, pl.ds(wid * BW, BW)], idx_v)

    def prep(s, nb):
        for j in range(NJ):
            sl = pl.ds(j * 16, 16)
            t = idx_v[s, sl]
            pidx_v[nb, sl] = lax.shift_right_logical(t, 1)
            cb_v[nb, sl] = lax.shift_left(jnp.bitwise_and(t, 1), 6)

    def issue_gather(s, nb):
        pltpu.async_copy(t2_hbm.at[pidx_v.at[nb]], g[nb], gsem[nb])
        pltpu.async_copy(pe_hbm.at[s], psplat_v.at[nb], gsem[nb])

    def wait_gather(b):
        pltpu.make_async_copy(t2_hbm.at[pl.ds(0, BW)], g[b], gsem[b]).wait()
        pltpu.make_async_copy(
            pe_hbm.at[0], psplat_v.at[b], gsem[b]
        ).wait()

    def issue_out(s, b):
        pltpu.async_copy(
            o[b], out_hbm.at[s, :, pl.ds(wid * BW, BW)], osem[b]
        )

    def wait_out(b):
        pltpu.make_async_copy(
            o[b], out_hbm.at[0, :, pl.ds(0, BW)], osem[b]
        ).wait()

    def transpose_add(gb, ob):
        rowj = []
        cbj = []
        for j in range(NJ):
            rowj.append(lax.iota(jnp.int32, 16) + j * 16)
            cbj.append(cb_v[gb, pl.ds(j * 16, 16)])

        @plsc.parallel_loop(0, DIM, step=1, unroll=8)
        def dbody(d):
            pe_sp = psplat_v[gb, pl.ds(d * 16, 16)]
            for j in range(NJ):
                val = plsc.load_gather(g[gb], [rowj[j], cbj[j] + d])
                o[ob][d, pl.ds(j * 16, 16)] = val + pe_sp

    for p in range(3):
        prep(p, p)
        issue_gather(p, p)

    def chunk_body(i4, _):
        for gb in range(4):
            s = i4 * 4 + gb
            ob = gb % 2

            @pl.when(s < S - 3)
            def _():
                prep(s + 3, (gb + 3) % 4)
                issue_gather(s + 3, (gb + 3) % 4)

            wait_gather(gb)

            @pl.when(s >= 2)
            def _():
                wait_out(ob)

            transpose_add(gb, ob)
            issue_out(s, ob)
        return 0

    lax.fori_loop(0, S // 4, chunk_body, 0)
    wait_out(0)
    wait_out(1)


def kernel(input, tok_table):
    idx_t = input.T.astype(jnp.int32)
    t2 = tok_table.reshape(VOCAB // 2, 2 * DIM)
    out_t = _emb(idx_t, t2, jnp.asarray(_PE_SPLAT))
    return out_t.transpose(2, 0, 1)


# speedup vs baseline: 1.0005x; 1.0005x over previous
import functools

import jax
import jax.numpy as jnp
import numpy as np
from jax import lax
from jax.experimental import pallas as pl
from jax.experimental.pallas import tpu as pltpu
from jax.experimental.pallas import tpu_sc as plsc

VOCAB = 1000000
DIM = 64
MAX_LEN = 256
B = 4096
S = 200

NUM_CORES = 2
NUM_SUBCORES = 16
NW = NUM_CORES * NUM_SUBCORES
BW = B // NW
NJ = BW // 16


def _sinusoidal_pe(max_len, dim):
    pos = np.arange(max_len, dtype=np.float32)[:, None]
    i = np.arange(0, dim, 2, dtype=np.float32)[None, :]
    angle = pos / np.power(10000.0, i / dim)
    pe = np.zeros((max_len, dim), dtype=np.float32)
    pe[:, 0::2] = np.sin(angle)
    pe[:, 1::2] = np.cos(angle)
    return pe


_PE = _sinusoidal_pe(MAX_LEN, DIM)[:S, :]
_PE_SPLAT = np.repeat(_PE, 16, axis=1).reshape(S, DIM * 16)


@functools.partial(
    pl.kernel,
    mesh=plsc.VectorSubcoreMesh(core_axis_name="c", subcore_axis_name="s"),
    out_type=jax.ShapeDtypeStruct((S, DIM, B), jnp.float32),
    compiler_params=pltpu.CompilerParams(
        use_tc_tiling_on_sc=True,
        needs_layout_passes=False,
        disable_bounds_checks=True,
    ),
    scratch_types=[
        pltpu.VMEM((S, BW), jnp.int32),
        pltpu.VMEM((4, BW), jnp.int32),
        pltpu.VMEM((4, BW), jnp.int32),
        pltpu.VMEM((BW, 128), jnp.float32),
        pltpu.VMEM((BW, 128), jnp.float32),
        pltpu.VMEM((BW, 128), jnp.float32),
        pltpu.VMEM((BW, 128), jnp.float32),
        pltpu.VMEM((DIM, BW), jnp.float32),
        pltpu.VMEM((DIM, BW), jnp.float32),
        pltpu.VMEM((4, DIM * 16), jnp.float32),
        pltpu.SemaphoreType.DMA,
        pltpu.SemaphoreType.DMA,
        pltpu.SemaphoreType.DMA,
        pltpu.SemaphoreType.DMA,
        pltpu.SemaphoreType.DMA,
        pltpu.SemaphoreType.DMA,
    ],
)
def _emb(idx_hbm, t2_hbm, pe_hbm, out_hbm,
         idx_v, pidx_v, cb_v, g0, g1, g2, g3, o0, o1, psplat_v,
         gsem0, gsem1, gsem2, gsem3, osem0, osem1):
    wid = lax.axis_index("s") * NUM_CORES + lax.axis_index("c")
    g = (g0, g1, g2, g3)
    o = (o0, o1)
    gsem = (gsem0, gsem1, gsem2, gsem3)
    osem = (osem0, osem1)

    pltpu.sync_copy(idx_hbm.at[:, pl.ds(wid * BW, BW)], idx_v)

    def prep(s, nb):
        for j in range(NJ):
            sl = pl.ds(j * 16, 16)
            t = idx_v[s, sl]
            pidx_v[nb, sl] = lax.shift_right_logical(t, 1)
            cb_v[nb, sl] = lax.shift_left(jnp.bitwise_and(t, 1), 6)

    def issue_gather(s, nb):
        pltpu.async_copy(t2_hbm.at[pidx_v.at[nb]], g[nb], gsem[nb])
        pltpu.async_copy(pe_hbm.at[s], psplat_v.at[nb], gsem[nb])

    def wait_gather(b):
        pltpu.make_async_copy(t2_hbm.at[pl.ds(0, BW)], g[b], gsem[b]).wait()
        pltpu.make_async_copy(
            pe_hbm.at[0], psplat_v.at[b], gsem[b]
        ).wait()

    def issue_out(s, b):
        pltpu.async_copy(
            o[b], out_hbm.at[s, :, pl.ds(wid * BW, BW)], osem[b]
        )

    def wait_out(b):
        pltpu.make_async_copy(
            o[b], out_hbm.at[0, :, pl.ds(0, BW)], osem[b]
        ).wait()

    def transpose_add(gb, ob):
        rowj = []
        cbj = []
        for j in range(NJ):
            rowj.append(lax.iota(jnp.int32, 16) + j * 16)
            cbj.append(cb_v[gb, pl.ds(j * 16, 16)])

        @plsc.parallel_loop(0, DIM, step=1, unroll=8)
        def dbody(d):
            pe_sp = psplat_v[gb, pl.ds(d * 16, 16)]
            for j in range(NJ):
                val = plsc.load_gather(g[gb], [rowj[j], cbj[j] + d])
                o[ob][d, pl.ds(j * 16, 16)] = val + pe_sp

    for p in range(3):
        prep(p, p)
        issue_gather(p, p)

    def chunk_body(i4, _):
        for gb in range(4):
            s = i4 * 4 + gb
            ob = gb % 2

            @pl.when(s < S - 3)
            def _():
                prep(s + 3, (gb + 3) % 4)
                issue_gather(s + 3, (gb + 3) % 4)

            wait_gather(gb)

            @pl.when(s >= 2)
            def _():
                wait_out(ob)

            transpose_add(gb, ob)
            issue_out(s, ob)
        return 0

    lax.fori_loop(0, S // 4, chunk_body, 0)
    wait_out(0)
    wait_out(1)


def kernel(input, tok_table):
    idx_t = input.T.astype(jnp.int32)
    t2 = tok_table.reshape(VOCAB // 2, 2 * DIM)
    out_t = _emb(idx_t, t2, jnp.asarray(_PE_SPLAT))
    return out_t.transpose(2, 0, 1)


# speedup vs baseline: 1.5157x; 1.5149x over previous
import functools

import jax
import jax.numpy as jnp
import numpy as np
from jax import lax
from jax.experimental import pallas as pl
from jax.experimental.pallas import tpu as pltpu
from jax.experimental.pallas import tpu_sc as plsc

VOCAB = 1000000
DIM = 64
MAX_LEN = 256
B = 4096
S = 200

NUM_CORES = 2
NUM_SUBCORES = 16
NW = NUM_CORES * NUM_SUBCORES
BW = B // NW
NJ = BW // 16


def _sinusoidal_pe(max_len, dim):
    pos = np.arange(max_len, dtype=np.float32)[:, None]
    i = np.arange(0, dim, 2, dtype=np.float32)[None, :]
    angle = pos / np.power(10000.0, i / dim)
    pe = np.zeros((max_len, dim), dtype=np.float32)
    pe[:, 0::2] = np.sin(angle)
    pe[:, 1::2] = np.cos(angle)
    return pe


_PE = _sinusoidal_pe(MAX_LEN, DIM)[:S, :]


@functools.partial(
    pl.kernel,
    mesh=plsc.VectorSubcoreMesh(core_axis_name="c", subcore_axis_name="s"),
    out_type=jax.ShapeDtypeStruct((S, DIM, B), jnp.float32),
    compiler_params=pltpu.CompilerParams(
        use_tc_tiling_on_sc=True,
        needs_layout_passes=False,
        disable_bounds_checks=True,
    ),
    scratch_types=[
        pltpu.VMEM((S, BW), jnp.int32),
        pltpu.VMEM((4, BW), jnp.int32),
        pltpu.VMEM((4, BW), jnp.int32),
        pltpu.VMEM((BW, 128), jnp.float32),
        pltpu.VMEM((BW, 128), jnp.float32),
        pltpu.VMEM((BW, 128), jnp.float32),
        pltpu.VMEM((BW, 128), jnp.float32),
        pltpu.VMEM((DIM, BW), jnp.float32),
        pltpu.VMEM((DIM, BW), jnp.float32),
        pltpu.VMEM((4, DIM), jnp.float32),
        pltpu.SemaphoreType.DMA,
        pltpu.SemaphoreType.DMA,
        pltpu.SemaphoreType.DMA,
        pltpu.SemaphoreType.DMA,
        pltpu.SemaphoreType.DMA,
        pltpu.SemaphoreType.DMA,
    ],
)
def _emb(idx_hbm, t2_hbm, pe_hbm, out_hbm,
         idx_v, pidx_v, cb_v, g0, g1, g2, g3, o0, o1, pe_v,
         gsem0, gsem1, gsem2, gsem3, osem0, osem1):
    wid = lax.axis_index("s") * NUM_CORES + lax.axis_index("c")
    g = (g0, g1, g2, g3)
    o = (o0, o1)
    gsem = (gsem0, gsem1, gsem2, gsem3)
    osem = (osem0, osem1)

    pltpu.sync_copy(idx_hbm.at[:, pl.ds(wid * BW, BW)], idx_v)

    def prep(s, nb):
        for j in range(NJ):
            sl = pl.ds(j * 16, 16)
            t = idx_v[s, sl]
            pidx_v[nb, sl] = lax.shift_right_logical(t, 1)
            cb_v[nb, sl] = lax.shift_left(jnp.bitwise_and(t, 1), 6)

    def issue_gather(s, nb):
        pltpu.async_copy(t2_hbm.at[pidx_v.at[nb]], g[nb], gsem[nb])
        pltpu.async_copy(pe_hbm.at[s], pe_v.at[nb], gsem[nb])

    def wait_gather(b):
        pltpu.make_async_copy(t2_hbm.at[pl.ds(0, BW)], g[b], gsem[b]).wait()
        pltpu.make_async_copy(pe_hbm.at[0], pe_v.at[b], gsem[b]).wait()

    def issue_out(s, b):
        pltpu.async_copy(
            o[b], out_hbm.at[s, :, pl.ds(wid * BW, BW)], osem[b]
        )

    def wait_out(b):
        pltpu.make_async_copy(
            o[b], out_hbm.at[0, :, pl.ds(0, BW)], osem[b]
        ).wait()

    def transpose_add(gb, ob):
        i16 = lax.iota(jnp.int32, 16)
        gbvec = jnp.full((16,), gb, jnp.int32)
        rowj = []
        cbj = []
        for j in range(NJ):
            rowj.append(i16 + j * 16)
            cbj.append(cb_v[gb, pl.ds(j * 16, 16)])

        @plsc.parallel_loop(0, DIM, step=1, unroll=4)
        def dbody(d):
            dcol = jnp.bitwise_and(i16 + d, DIM - 1)
            prot = plsc.load_gather(pe_v, [gbvec, dcol])
            for j in range(NJ):
                val = plsc.load_gather(g[gb], [rowj[j], cbj[j] + dcol])
                plsc.store_scatter(o[ob], [dcol, rowj[j]], val + prot)

    for p in range(3):
        prep(p, p)
        issue_gather(p, p)

    def chunk_body(i4, _):
        for gb in range(4):
            s = i4 * 4 + gb
            ob = gb % 2

            @pl.when(s < S - 3)
            def _():
                prep(s + 3, (gb + 3) % 4)
                issue_gather(s + 3, (gb + 3) % 4)

            wait_gather(gb)

            @pl.when(s >= 2)
            def _():
                wait_out(ob)

            transpose_add(gb, ob)
            issue_out(s, ob)
        return 0

    lax.fori_loop(0, S // 4, chunk_body, 0)
    wait_out(0)
    wait_out(1)


def kernel(input, tok_table):
    idx_t = input.T.astype(jnp.int32)
    t2 = tok_table.reshape(VOCAB // 2, 2 * DIM)
    out_t = _emb(idx_t, t2, jnp.asarray(_PE))
    return out_t.transpose(2, 0, 1)


# speedup vs baseline: 2.7194x; 1.7942x over previous
import functools

import jax
import jax.numpy as jnp
import numpy as np
from jax import lax
from jax.experimental import pallas as pl
from jax.experimental.pallas import tpu as pltpu
from jax.experimental.pallas import tpu_sc as plsc

VOCAB = 1000000
DIM = 64
MAX_LEN = 256
B = 4096
S = 200

NUM_CORES = 2
NUM_SUBCORES = 16
NW = NUM_CORES * NUM_SUBCORES
BW = B // NW
NJ = BW // 16


def _sinusoidal_pe(max_len, dim):
    pos = np.arange(max_len, dtype=np.float32)[:, None]
    i = np.arange(0, dim, 2, dtype=np.float32)[None, :]
    angle = pos / np.power(10000.0, i / dim)
    pe = np.zeros((max_len, dim), dtype=np.float32)
    pe[:, 0::2] = np.sin(angle)
    pe[:, 1::2] = np.cos(angle)
    return pe


_PE = _sinusoidal_pe(MAX_LEN, DIM)[:S, :]


NQ = VOCAB // 128
QTAIL = NQ * 128


@functools.partial(
    pl.kernel,
    mesh=plsc.VectorSubcoreMesh(core_axis_name="c", subcore_axis_name="s"),
    out_type=jax.ShapeDtypeStruct((VOCAB // 2, 2 * DIM), jnp.float32),
    compiler_params=pltpu.CompilerParams(
        use_tc_tiling_on_sc=True,
        needs_layout_passes=False,
        disable_bounds_checks=True,
    ),
    scratch_types=[
        pltpu.VMEM((DIM, 128), jnp.float32),
        pltpu.VMEM((DIM, 128), jnp.float32),
        pltpu.VMEM((DIM, 128), jnp.float32),
        pltpu.VMEM((DIM, 128), jnp.float32),
        pltpu.VMEM((32, 128), jnp.float32),
        pltpu.SemaphoreType.DMA,
        pltpu.SemaphoreType.DMA,
        pltpu.SemaphoreType.DMA,
        pltpu.SemaphoreType.DMA,
    ],
)
def _relayout(tt_hbm, tail_hbm, t2_hbm,
              bd0, bd1, pb0, pb1, tail_v, rsem0, rsem1, wsem0, wsem1):
    wid = lax.axis_index("s") * NUM_CORES + lax.axis_index("c")
    bd = (bd0, bd1)
    pb = (pb0, pb1)
    rsem = (rsem0, rsem1)
    wsem = (wsem0, wsem1)

    def issue_read(k, b):
        q = wid + NW * k
        pltpu.async_copy(tt_hbm.at[:, pl.ds(128 * q, 128)], bd[b], rsem[b])

    def wait_read(b):
        pltpu.make_async_copy(tt_hbm.at[:, pl.ds(0, 128)], bd[b], rsem[b]).wait()

    def issue_write(k, b):
        q = wid + NW * k
        pltpu.async_copy(pb[b], t2_hbm.at[pl.ds(64 * q, 64)], wsem[b])

    def wait_write(b):
        pltpu.make_async_copy(pb[b], t2_hbm.at[pl.ds(0, 64)], wsem[b]).wait()

    def transpose(b):
        i16 = lax.iota(jnp.int32, 16)
        rowj = [i16 + 16 * j for j in range(4)]
        cnst = [32 * j + 2 * i16 for j in range(4)]

        @plsc.parallel_loop(0, 128, step=1, unroll=4)
        def cbody(c0):
            colv = jnp.bitwise_and(i16 + c0, 127)
            d = jnp.bitwise_and(colv, DIM - 1)
            u = lax.shift_right_logical(colv, 6)
            for j in range(4):
                val = plsc.load_gather(bd[b], [d, cnst[j] + u])
                plsc.store_scatter(pb[b], [rowj[j], colv], val)

    issue_read(0, 0)
    issue_read(1, 1)

    def body(k2, _):
        for b in range(2):
            k = 2 * k2 + b
            q = wid + NW * k

            @pl.when(q < NQ)
            def _():
                wait_read(b)

                @pl.when(k >= 2)
                def _():
                    wait_write(b)

                transpose(b)
                issue_write(k, b)

                @pl.when(q + 2 * NW < NQ)
                def _():
                    issue_read(k + 2, b)
        return 0

    lax.fori_loop(0, 123, body, 0)
    wait_write(0)
    wait_write(1)

    @pl.when(wid == 0)
    def _():
        pltpu.sync_copy(tail_hbm, tail_v)
        pltpu.sync_copy(tail_v, t2_hbm.at[pl.ds(QTAIL // 2, 32)])


@functools.partial(
    pl.kernel,
    mesh=plsc.VectorSubcoreMesh(core_axis_name="c", subcore_axis_name="s"),
    out_type=jax.ShapeDtypeStruct((S, DIM, B), jnp.float32),
    compiler_params=pltpu.CompilerParams(
        use_tc_tiling_on_sc=True,
        needs_layout_passes=False,
        disable_bounds_checks=True,
    ),
    scratch_types=[
        pltpu.VMEM((S, BW), jnp.int32),
        pltpu.VMEM((4, BW), jnp.int32),
        pltpu.VMEM((4, BW), jnp.int32),
        pltpu.VMEM((BW, 128), jnp.float32),
        pltpu.VMEM((BW, 128), jnp.float32),
        pltpu.VMEM((BW, 128), jnp.float32),
        pltpu.VMEM((BW, 128), jnp.float32),
        pltpu.VMEM((DIM, BW), jnp.float32),
        pltpu.VMEM((DIM, BW), jnp.float32),
        pltpu.VMEM((4, DIM), jnp.float32),
        pltpu.SemaphoreType.DMA,
        pltpu.SemaphoreType.DMA,
        pltpu.SemaphoreType.DMA,
        pltpu.SemaphoreType.DMA,
        pltpu.SemaphoreType.DMA,
        pltpu.SemaphoreType.DMA,
    ],
)
def _emb(idx_hbm, t2_hbm, pe_hbm, out_hbm,
         idx_v, pidx_v, cb_v, g0, g1, g2, g3, o0, o1, pe_v,
         gsem0, gsem1, gsem2, gsem3, osem0, osem1):
    wid = lax.axis_index("s") * NUM_CORES + lax.axis_index("c")
    g = (g0, g1, g2, g3)
    o = (o0, o1)
    gsem = (gsem0, gsem1, gsem2, gsem3)
    osem = (osem0, osem1)

    pltpu.sync_copy(idx_hbm.at[:, pl.ds(wid * BW, BW)], idx_v)

    def prep(s, nb):
        for j in range(NJ):
            sl = pl.ds(j * 16, 16)
            t = idx_v[s, sl]
            pidx_v[nb, sl] = lax.shift_right_logical(t, 1)
            cb_v[nb, sl] = lax.shift_left(jnp.bitwise_and(t, 1), 6)

    def issue_gather(s, nb):
        pltpu.async_copy(t2_hbm.at[pidx_v.at[nb]], g[nb], gsem[nb])
        pltpu.async_copy(pe_hbm.at[s], pe_v.at[nb], gsem[nb])

    def wait_gather(b):
        pltpu.make_async_copy(t2_hbm.at[pl.ds(0, BW)], g[b], gsem[b]).wait()
        pltpu.make_async_copy(pe_hbm.at[0], pe_v.at[b], gsem[b]).wait()

    def issue_out(s, b):
        pltpu.async_copy(
            o[b], out_hbm.at[s, :, pl.ds(wid * BW, BW)], osem[b]
        )

    def wait_out(b):
        pltpu.make_async_copy(
            o[b], out_hbm.at[0, :, pl.ds(0, BW)], osem[b]
        ).wait()

    def transpose_add(gb, ob):
        i16 = lax.iota(jnp.int32, 16)
        gbvec = jnp.full((16,), gb, jnp.int32)
        rowj = []
        cbj = []
        for j in range(NJ):
            rowj.append(i16 + j * 16)
            cbj.append(cb_v[gb, pl.ds(j * 16, 16)])

        @plsc.parallel_loop(0, DIM, step=1, unroll=4)
        def dbody(d):
            dcol = jnp.bitwise_and(i16 + d, DIM - 1)
            prot = plsc.load_gather(pe_v, [gbvec, dcol])
            for j in range(NJ):
                val = plsc.load_gather(g[gb], [rowj[j], cbj[j] + dcol])
                plsc.store_scatter(o[ob], [dcol, rowj[j]], val + prot)

    for p in range(3):
        prep(p, p)
        issue_gather(p, p)

    def chunk_body(i4, _):
        for gb in range(4):
            s = i4 * 4 + gb
            ob = gb % 2

            @pl.when(s < S - 3)
            def _():
                prep(s + 3, (gb + 3) % 4)
                issue_gather(s + 3, (gb + 3) % 4)

            wait_gather(gb)

            @pl.when(s >= 2)
            def _():
                wait_out(ob)

            transpose_add(gb, ob)
            issue_out(s, ob)
        return 0

    lax.fori_loop(0, S // 4, chunk_body, 0)
    wait_out(0)
    wait_out(1)


def kernel(input, tok_table):
    idx_t = input.T.astype(jnp.int32)
    tt = tok_table.T
    tail = tok_table[QTAIL:, :].reshape(32, 128)
    t2 = _relayout(tt, tail)
    out_t = _emb(idx_t, t2, jnp.asarray(_PE))
    return out_t.transpose(2, 0, 1)
